# Initial kernel scaffold; baseline (speedup 1.0000x reference)
import jax
import jax.numpy as jnp
from jax.experimental import pallas as pl


def kernel(x, edge_index, W1, b1, W2, b2):
    raise NotImplementedError("write your pallas kernel here")


# speedup vs baseline: 27.4004x; 27.4004x over previous
import jax
import jax.numpy as jnp
from jax import lax
from jax.experimental import pallas as pl
from jax.experimental.pallas import tpu as pltpu
from jax.experimental.pallas import tpu_sc as plsc

N_NODES = 10000
N_PAD = 10240
IN_CH = 128
HID = 256
OUT_CH = 128
N_EDGES = 320000

NC = 2
NS = 16
NW = NC * NS
E_PER_W = N_EDGES // NW
CHUNK = 125
NCHUNK = E_PER_W // CHUNK
STRIPE = N_PAD // NS
ZROWS = STRIPE // 16
DEGW = 16
ACC_W = 64

_mesh = plsc.VectorSubcoreMesh(core_axis_name="c", subcore_axis_name="s")
_sc_params = pltpu.CompilerParams(use_tc_tiling_on_sc=False)


def _fill_const(ref, rows, width, value):
    lanes = width // 16

    def body(i, _):
        r = i // lanes
        k = i % lanes
        ref[r, pl.ds(k * 16, 16)] = jnp.full((16,), value, jnp.float32)
        return 0

    lax.fori_loop(0, rows * lanes, body, 0)


def _zero_stripe(zsrc, acc, s):
    for q in range(STRIPE // ZROWS):
        pltpu.sync_copy(
            zsrc.at[pl.ds(0, ZROWS)],
            acc.at[pl.ds(s * STRIPE + q * ZROWS, ZROWS)],
        )


def _deg_body(dst_hbm, out_hbm, dst_v, ones_v, ssem, acc):
    c = lax.axis_index("c")
    s = lax.axis_index("s")
    wid = s * NC + c

    _fill_const(ones_v, CHUNK, DEGW, 0.0)
    _zero_stripe(ones_v, acc, s)
    _fill_const(ones_v, CHUNK, DEGW, 1.0)
    plsc.subcore_barrier()

    pltpu.sync_copy(dst_hbm.at[wid], dst_v)

    depth = 4
    for q in range(depth):
        pltpu.async_copy(ones_v, acc.at[dst_v.at[q]], ssem, add=True)

    def body(i, _):
        pltpu.make_async_copy(ones_v, acc.at[dst_v.at[i]], ssem).wait()
        pltpu.async_copy(ones_v, acc.at[dst_v.at[i + depth]], ssem, add=True)
        return 0

    lax.fori_loop(0, NCHUNK - depth, body, 0)
    for q in range(depth):
        pltpu.make_async_copy(ones_v, acc.at[dst_v.at[q]], ssem).wait()

    plsc.subcore_barrier()
    pltpu.sync_copy(
        acc.at[pl.ds(s * STRIPE, STRIPE)],
        out_hbm.at[c, pl.ds(s * STRIPE, STRIPE)],
    )


_deg_call = pl.kernel(
    _deg_body,
    out_type=jax.ShapeDtypeStruct((NC, N_PAD, DEGW), jnp.float32),
    mesh=_mesh,
    scratch_types=[
        pltpu.VMEM((NCHUNK, CHUNK), jnp.int32),
        pltpu.VMEM((CHUNK, DEGW), jnp.float32),
        pltpu.SemaphoreType.DMA,
        pltpu.VMEM_SHARED((N_PAD, DEGW), jnp.float32),
    ],
    compiler_params=_sc_params,
)


def _agg_body(table_hbm, src_hbm, dst_hbm, out_hbm,
              src_v, dst_v, rows0, rows1, g0, g1, acc):
    c = lax.axis_index("c")
    s = lax.axis_index("s")
    wid = s * NC + c

    _fill_const(rows0, CHUNK, ACC_W, 0.0)
    _zero_stripe(rows0, acc, s)
    plsc.subcore_barrier()

    pltpu.sync_copy(src_hbm.at[wid], src_v)
    pltpu.sync_copy(dst_hbm.at[wid], dst_v)

    pltpu.async_copy(table_hbm.at[src_v.at[0]], rows0, g0)

    def body(i, _):
        j0 = 2 * i
        pltpu.async_copy(table_hbm.at[src_v.at[j0 + 1]], rows1, g1)
        pltpu.make_async_copy(table_hbm.at[src_v.at[j0]], rows0, g0).wait()
        pltpu.sync_copy(rows0, acc.at[dst_v.at[j0]], add=True)
        pltpu.async_copy(table_hbm.at[src_v.at[j0 + 2]], rows0, g0)
        pltpu.make_async_copy(table_hbm.at[src_v.at[j0]], rows1, g1).wait()
        pltpu.sync_copy(rows1, acc.at[dst_v.at[j0 + 1]], add=True)
        return 0

    lax.fori_loop(0, (NCHUNK - 2) // 2, body, 0)
    pltpu.async_copy(table_hbm.at[src_v.at[NCHUNK - 1]], rows1, g1)
    pltpu.make_async_copy(table_hbm.at[src_v.at[0]], rows0, g0).wait()
    pltpu.sync_copy(rows0, acc.at[dst_v.at[NCHUNK - 2]], add=True)
    pltpu.make_async_copy(table_hbm.at[src_v.at[0]], rows1, g1).wait()
    pltpu.sync_copy(rows1, acc.at[dst_v.at[NCHUNK - 1]], add=True)

    plsc.subcore_barrier()
    pltpu.sync_copy(
        acc.at[pl.ds(s * STRIPE, STRIPE)],
        out_hbm.at[c, pl.ds(s * STRIPE, STRIPE)],
    )


_agg_call = pl.kernel(
    _agg_body,
    out_type=jax.ShapeDtypeStruct((NC, N_PAD, ACC_W), jnp.float32),
    mesh=_mesh,
    scratch_types=[
        pltpu.VMEM((NCHUNK, CHUNK), jnp.int32),
        pltpu.VMEM((NCHUNK, CHUNK), jnp.int32),
        pltpu.VMEM((CHUNK, ACC_W), jnp.float32),
        pltpu.VMEM((CHUNK, ACC_W), jnp.float32),
        pltpu.SemaphoreType.DMA,
        pltpu.SemaphoreType.DMA,
        pltpu.VMEM_SHARED((N_PAD, ACC_W), jnp.float32),
    ],
    compiler_params=_sc_params,
)


ROWS_BLK = 1000
GRID = N_NODES // ROWS_BLK


def _dis_block(deg_ref):
    d = deg_ref[0, :, 0:1] + deg_ref[1, :, 0:1] + 1.0
    return lax.rsqrt(d)


def _scale_body(deg_ref, x_ref, lo_ref, hi_ref):
    dis = _dis_block(deg_ref)
    sc = x_ref[...] * dis
    lo_ref[...] = sc[:, :ACC_W]
    hi_ref[...] = sc[:, ACC_W:]


_scale_call = pl.pallas_call(
    _scale_body,
    grid=(GRID,),
    in_specs=[
        pl.BlockSpec((NC, ROWS_BLK, DEGW), lambda i: (0, i, 0)),
        pl.BlockSpec((ROWS_BLK, IN_CH), lambda i: (i, 0)),
    ],
    out_specs=[
        pl.BlockSpec((ROWS_BLK, ACC_W), lambda i: (i, 0)),
        pl.BlockSpec((ROWS_BLK, ACC_W), lambda i: (i, 0)),
    ],
    out_shape=[
        jax.ShapeDtypeStruct((N_NODES, ACC_W), jnp.float32),
        jax.ShapeDtypeStruct((N_NODES, ACC_W), jnp.float32),
    ],
)


def _mid_body(deg_ref, alo_ref, ahi_ref, slo_ref, shi_ref,
              w1_ref, b1_ref, w2_ref, olo_ref, ohi_ref):
    dis = _dis_block(deg_ref)
    a_lo = (alo_ref[0, :, :] + alo_ref[1, :, :] + slo_ref[...]) * dis
    a_hi = (ahi_ref[0, :, :] + ahi_ref[1, :, :] + shi_ref[...]) * dis
    a = jnp.concatenate([a_lo, a_hi], axis=1)
    h = jnp.dot(a, w1_ref[...], preferred_element_type=jnp.float32)
    h = jnp.maximum(h + b1_ref[...], 0.0)
    t = jnp.dot(h, w2_ref[...], preferred_element_type=jnp.float32) * dis
    olo_ref[...] = t[:, :ACC_W]
    ohi_ref[...] = t[:, ACC_W:]


_mid_call = pl.pallas_call(
    _mid_body,
    grid=(GRID,),
    in_specs=[
        pl.BlockSpec((NC, ROWS_BLK, DEGW), lambda i: (0, i, 0)),
        pl.BlockSpec((NC, ROWS_BLK, ACC_W), lambda i: (0, i, 0)),
        pl.BlockSpec((NC, ROWS_BLK, ACC_W), lambda i: (0, i, 0)),
        pl.BlockSpec((ROWS_BLK, ACC_W), lambda i: (i, 0)),
        pl.BlockSpec((ROWS_BLK, ACC_W), lambda i: (i, 0)),
        pl.BlockSpec((IN_CH, HID), lambda i: (0, 0)),
        pl.BlockSpec((1, HID), lambda i: (0, 0)),
        pl.BlockSpec((HID, OUT_CH), lambda i: (0, 0)),
    ],
    out_specs=[
        pl.BlockSpec((ROWS_BLK, ACC_W), lambda i: (i, 0)),
        pl.BlockSpec((ROWS_BLK, ACC_W), lambda i: (i, 0)),
    ],
    out_shape=[
        jax.ShapeDtypeStruct((N_NODES, ACC_W), jnp.float32),
        jax.ShapeDtypeStruct((N_NODES, ACC_W), jnp.float32),
    ],
)


def _out_body(deg_ref, alo_ref, ahi_ref, slo_ref, shi_ref, b2_ref, o_ref):
    dis = _dis_block(deg_ref)
    o_lo = (alo_ref[0, :, :] + alo_ref[1, :, :] + slo_ref[...]) * dis
    o_hi = (ahi_ref[0, :, :] + ahi_ref[1, :, :] + shi_ref[...]) * dis
    o_ref[...] = jnp.concatenate([o_lo, o_hi], axis=1) + b2_ref[...]


_out_call = pl.pallas_call(
    _out_body,
    grid=(GRID,),
    in_specs=[
        pl.BlockSpec((NC, ROWS_BLK, DEGW), lambda i: (0, i, 0)),
        pl.BlockSpec((NC, ROWS_BLK, ACC_W), lambda i: (0, i, 0)),
        pl.BlockSpec((NC, ROWS_BLK, ACC_W), lambda i: (0, i, 0)),
        pl.BlockSpec((ROWS_BLK, ACC_W), lambda i: (i, 0)),
        pl.BlockSpec((ROWS_BLK, ACC_W), lambda i: (i, 0)),
        pl.BlockSpec((1, OUT_CH), lambda i: (0, 0)),
    ],
    out_specs=pl.BlockSpec((ROWS_BLK, OUT_CH), lambda i: (i, 0)),
    out_shape=jax.ShapeDtypeStruct((N_NODES, OUT_CH), jnp.float32),
)


@jax.jit
def kernel(x, edge_index, W1, b1, W2, b2):
    ei = edge_index.astype(jnp.int32)
    src2 = ei[0].reshape(NW, NCHUNK, CHUNK)
    dst2 = ei[1].reshape(NW, NCHUNK, CHUNK)

    degp = _deg_call(dst2)
    s1lo, s1hi = _scale_call(degp, x)
    a1lo = _agg_call(s1lo, src2, dst2)
    a1hi = _agg_call(s1hi, src2, dst2)
    s2lo, s2hi = _mid_call(degp, a1lo, a1hi, s1lo, s1hi,
                           W1, b1.reshape(1, HID), W2)
    a2lo = _agg_call(s2lo, src2, dst2)
    a2hi = _agg_call(s2hi, src2, dst2)
    return _out_call(degp, a2lo, a2hi, s2lo, s2hi, b2.reshape(1, OUT_CH))


# speedup vs baseline: 29.5439x; 1.0782x over previous
import jax
import jax.numpy as jnp
from jax import lax
from jax.experimental import pallas as pl
from jax.experimental.pallas import tpu as pltpu
from jax.experimental.pallas import tpu_sc as plsc

N_NODES = 10000
N_PAD = 10240
IN_CH = 128
HID = 256
OUT_CH = 128
N_EDGES = 320000

NC = 2
NS = 16
E_PER_T = N_EDGES // NS
CHUNK = 125
NCHUNK = E_PER_T // CHUNK
DCHUNK = NCHUNK // NC
STRIPE = N_PAD // NS
ZROWS = STRIPE // 16
DEGW = 16
ACC_W = 64

_mesh = plsc.VectorSubcoreMesh(core_axis_name="c", subcore_axis_name="s")
_sc_params = pltpu.CompilerParams(use_tc_tiling_on_sc=False)


def _fill_const(ref, rows, width, value):
    lanes = width // 16

    def body(i, _):
        r = i // lanes
        k = i % lanes
        ref[r, pl.ds(k * 16, 16)] = jnp.full((16,), value, jnp.float32)
        return 0

    lax.fori_loop(0, rows * lanes, body, 0)


def _zero_stripe(zsrc, acc, s, zsem):
    n = STRIPE // ZROWS
    for q in range(n):
        pltpu.async_copy(
            zsrc.at[pl.ds(0, ZROWS)],
            acc.at[pl.ds(s * STRIPE + q * ZROWS, ZROWS)],
            zsem,
        )
    for q in range(n):
        pltpu.make_async_copy(
            zsrc.at[pl.ds(0, ZROWS)],
            acc.at[pl.ds(s * STRIPE, ZROWS)],
            zsem,
        ).wait()


def _deg_body(dst_hbm, out_hbm, dst_v, ones_v, ssem, acc):
    c = lax.axis_index("c")
    s = lax.axis_index("s")

    _fill_const(ones_v, CHUNK, DEGW, 0.0)
    _zero_stripe(ones_v, acc, s, ssem)
    _fill_const(ones_v, CHUNK, DEGW, 1.0)
    plsc.subcore_barrier()

    pltpu.sync_copy(dst_hbm.at[s, pl.ds(c * DCHUNK, DCHUNK)], dst_v)

    depth = 4
    for q in range(depth):
        pltpu.async_copy(ones_v, acc.at[dst_v.at[q]], ssem, add=True)

    def body(i, _):
        pltpu.make_async_copy(ones_v, acc.at[dst_v.at[i]], ssem).wait()
        pltpu.async_copy(ones_v, acc.at[dst_v.at[i + depth]], ssem, add=True)
        return 0

    lax.fori_loop(0, DCHUNK - depth, body, 0)
    for q in range(depth):
        pltpu.make_async_copy(ones_v, acc.at[dst_v.at[q]], ssem).wait()

    plsc.subcore_barrier()
    pltpu.sync_copy(
        acc.at[pl.ds(s * STRIPE, STRIPE)],
        out_hbm.at[c, pl.ds(s * STRIPE, STRIPE)],
    )


_deg_call = pl.kernel(
    _deg_body,
    out_type=jax.ShapeDtypeStruct((NC, N_PAD, DEGW), jnp.float32),
    mesh=_mesh,
    scratch_types=[
        pltpu.VMEM((DCHUNK, CHUNK), jnp.int32),
        pltpu.VMEM((CHUNK, DEGW), jnp.float32),
        pltpu.SemaphoreType.DMA,
        pltpu.VMEM_SHARED((N_PAD, DEGW), jnp.float32),
    ],
    compiler_params=_sc_params,
)


def _agg_body(table_hbm, src_hbm, dst_hbm, out_hbm,
              src_v, dst_v, rows0, rows1, g0, g1, acc):
    c = lax.axis_index("c")
    s = lax.axis_index("s")

    _fill_const(rows0, CHUNK, ACC_W, 0.0)
    _zero_stripe(rows0, acc, s, g0)
    plsc.subcore_barrier()

    pltpu.sync_copy(src_hbm.at[s], src_v)
    pltpu.sync_copy(dst_hbm.at[s], dst_v)

    table = table_hbm.at[c]

    pltpu.async_copy(table.at[src_v.at[0]], rows0, g0)

    def body(i, _):
        j0 = 2 * i
        pltpu.async_copy(table.at[src_v.at[j0 + 1]], rows1, g1)
        pltpu.make_async_copy(table.at[src_v.at[j0]], rows0, g0).wait()
        pltpu.sync_copy(rows0, acc.at[dst_v.at[j0]], add=True)
        pltpu.async_copy(table.at[src_v.at[j0 + 2]], rows0, g0)
        pltpu.make_async_copy(table.at[src_v.at[j0]], rows1, g1).wait()
        pltpu.sync_copy(rows1, acc.at[dst_v.at[j0 + 1]], add=True)
        return 0

    lax.fori_loop(0, (NCHUNK - 2) // 2, body, 0)
    pltpu.async_copy(table.at[src_v.at[NCHUNK - 1]], rows1, g1)
    pltpu.make_async_copy(table.at[src_v.at[0]], rows0, g0).wait()
    pltpu.sync_copy(rows0, acc.at[dst_v.at[NCHUNK - 2]], add=True)
    pltpu.make_async_copy(table.at[src_v.at[0]], rows1, g1).wait()
    pltpu.sync_copy(rows1, acc.at[dst_v.at[NCHUNK - 1]], add=True)

    plsc.subcore_barrier()
    pltpu.sync_copy(
        acc.at[pl.ds(s * STRIPE, STRIPE)],
        out_hbm.at[c, pl.ds(s * STRIPE, STRIPE)],
    )


_agg_call = pl.kernel(
    _agg_body,
    out_type=jax.ShapeDtypeStruct((NC, N_PAD, ACC_W), jnp.float32),
    mesh=_mesh,
    scratch_types=[
        pltpu.VMEM((NCHUNK, CHUNK), jnp.int32),
        pltpu.VMEM((NCHUNK, CHUNK), jnp.int32),
        pltpu.VMEM((CHUNK, ACC_W), jnp.float32),
        pltpu.VMEM((CHUNK, ACC_W), jnp.float32),
        pltpu.SemaphoreType.DMA,
        pltpu.SemaphoreType.DMA,
        pltpu.VMEM_SHARED((N_PAD, ACC_W), jnp.float32),
    ],
    compiler_params=_sc_params,
)


ROWS_BLK = 1000
GRID = N_NODES // ROWS_BLK


def _dis_block(deg_ref):
    d = deg_ref[0, :, 0:1] + deg_ref[1, :, 0:1] + 1.0
    return lax.rsqrt(d)


def _scale_body(deg_ref, x_ref, o_ref):
    dis = _dis_block(deg_ref)
    sc = x_ref[...] * dis
    o_ref[0, :, :] = sc[:, :ACC_W]
    o_ref[1, :, :] = sc[:, ACC_W:]


_scale_call = pl.pallas_call(
    _scale_body,
    grid=(GRID,),
    in_specs=[
        pl.BlockSpec((NC, ROWS_BLK, DEGW), lambda i: (0, i, 0)),
        pl.BlockSpec((ROWS_BLK, IN_CH), lambda i: (i, 0)),
    ],
    out_specs=pl.BlockSpec((NC, ROWS_BLK, ACC_W), lambda i: (0, i, 0)),
    out_shape=jax.ShapeDtypeStruct((NC, N_NODES, ACC_W), jnp.float32),
)


def _mid_body(deg_ref, agg_ref, s1_ref, w1_ref, b1_ref, w2_ref, o_ref):
    dis = _dis_block(deg_ref)
    a_lo = (agg_ref[0, :, :] + s1_ref[0, :, :]) * dis
    a_hi = (agg_ref[1, :, :] + s1_ref[1, :, :]) * dis
    a = jnp.concatenate([a_lo, a_hi], axis=1)
    h = jnp.dot(a, w1_ref[...], preferred_element_type=jnp.float32)
    h = jnp.maximum(h + b1_ref[...], 0.0)
    t = jnp.dot(h, w2_ref[...], preferred_element_type=jnp.float32) * dis
    o_ref[0, :, :] = t[:, :ACC_W]
    o_ref[1, :, :] = t[:, ACC_W:]


_mid_call = pl.pallas_call(
    _mid_body,
    grid=(GRID,),
    in_specs=[
        pl.BlockSpec((NC, ROWS_BLK, DEGW), lambda i: (0, i, 0)),
        pl.BlockSpec((NC, ROWS_BLK, ACC_W), lambda i: (0, i, 0)),
        pl.BlockSpec((NC, ROWS_BLK, ACC_W), lambda i: (0, i, 0)),
        pl.BlockSpec((IN_CH, HID), lambda i: (0, 0)),
        pl.BlockSpec((1, HID), lambda i: (0, 0)),
        pl.BlockSpec((HID, OUT_CH), lambda i: (0, 0)),
    ],
    out_specs=pl.BlockSpec((NC, ROWS_BLK, ACC_W), lambda i: (0, i, 0)),
    out_shape=jax.ShapeDtypeStruct((NC, N_NODES, ACC_W), jnp.float32),
)


def _out_body(deg_ref, agg_ref, s2_ref, b2_ref, o_ref):
    dis = _dis_block(deg_ref)
    o_lo = (agg_ref[0, :, :] + s2_ref[0, :, :]) * dis
    o_hi = (agg_ref[1, :, :] + s2_ref[1, :, :]) * dis
    o_ref[...] = jnp.concatenate([o_lo, o_hi], axis=1) + b2_ref[...]


_out_call = pl.pallas_call(
    _out_body,
    grid=(GRID,),
    in_specs=[
        pl.BlockSpec((NC, ROWS_BLK, DEGW), lambda i: (0, i, 0)),
        pl.BlockSpec((NC, ROWS_BLK, ACC_W), lambda i: (0, i, 0)),
        pl.BlockSpec((NC, ROWS_BLK, ACC_W), lambda i: (0, i, 0)),
        pl.BlockSpec((1, OUT_CH), lambda i: (0, 0)),
    ],
    out_specs=pl.BlockSpec((ROWS_BLK, OUT_CH), lambda i: (i, 0)),
    out_shape=jax.ShapeDtypeStruct((N_NODES, OUT_CH), jnp.float32),
)


@jax.jit
def kernel(x, edge_index, W1, b1, W2, b2):
    ei = edge_index.astype(jnp.int32)
    src2 = ei[0].reshape(NS, NCHUNK, CHUNK)
    dst2 = ei[1].reshape(NS, NCHUNK, CHUNK)

    degp = _deg_call(dst2)
    s1 = _scale_call(degp, x)
    a1 = _agg_call(s1, src2, dst2)
    s2 = _mid_call(degp, a1, s1, W1, b1.reshape(1, HID), W2)
    a2 = _agg_call(s2, src2, dst2)
    return _out_call(degp, a2, s2, b2.reshape(1, OUT_CH))


# speedup vs baseline: 30.6314x; 1.0368x over previous
import jax
import jax.numpy as jnp
from jax import lax
from jax.experimental import pallas as pl
from jax.experimental.pallas import tpu as pltpu
from jax.experimental.pallas import tpu_sc as plsc

N_NODES = 10000
N_PAD = 10240
IN_CH = 128
HID = 256
OUT_CH = 128
N_EDGES = 320000

NC = 2
NS = 16
E_PER_T = N_EDGES // NS
CHUNK = 125
NCHUNK = E_PER_T // CHUNK
DCHUNK = NCHUNK // NC
STRIPE = N_PAD // NS
ZROWS = STRIPE // 16
DEGW = 16
ACC_W = 64

_mesh = plsc.VectorSubcoreMesh(core_axis_name="c", subcore_axis_name="s")
_sc_params = pltpu.CompilerParams(use_tc_tiling_on_sc=False)


def _fill_const(ref, rows, width, value):
    lanes = width // 16

    def body(i, _):
        r = i // lanes
        k = i % lanes
        ref[r, pl.ds(k * 16, 16)] = jnp.full((16,), value, jnp.float32)
        return 0

    lax.fori_loop(0, rows * lanes, body, 0)


def _zero_stripe(zsrc, acc, s, zsem):
    n = STRIPE // ZROWS
    for q in range(n):
        pltpu.async_copy(
            zsrc.at[pl.ds(0, ZROWS)],
            acc.at[pl.ds(s * STRIPE + q * ZROWS, ZROWS)],
            zsem,
        )
    for q in range(n):
        pltpu.make_async_copy(
            zsrc.at[pl.ds(0, ZROWS)],
            acc.at[pl.ds(s * STRIPE, ZROWS)],
            zsem,
        ).wait()


def _deg_body(dst_hbm, out_hbm, dst_v, ones_v, ssem, acc):
    c = lax.axis_index("c")
    s = lax.axis_index("s")

    _fill_const(ones_v, CHUNK, DEGW, 0.0)
    _zero_stripe(ones_v, acc, s, ssem)
    _fill_const(ones_v, CHUNK, DEGW, 1.0)
    plsc.subcore_barrier()

    pltpu.sync_copy(dst_hbm.at[s, pl.ds(c * DCHUNK, DCHUNK)], dst_v)

    depth = 4
    for q in range(depth):
        pltpu.async_copy(ones_v, acc.at[dst_v.at[q]], ssem, add=True)

    def body(i, _):
        pltpu.make_async_copy(ones_v, acc.at[dst_v.at[i]], ssem).wait()
        pltpu.async_copy(ones_v, acc.at[dst_v.at[i + depth]], ssem, add=True)
        return 0

    lax.fori_loop(0, DCHUNK - depth, body, 0)
    for q in range(depth):
        pltpu.make_async_copy(ones_v, acc.at[dst_v.at[q]], ssem).wait()

    plsc.subcore_barrier()
    pltpu.sync_copy(
        acc.at[pl.ds(s * STRIPE, STRIPE)],
        out_hbm.at[c, pl.ds(s * STRIPE, STRIPE)],
    )


_deg_call = pl.kernel(
    _deg_body,
    out_type=jax.ShapeDtypeStruct((NC, N_PAD, DEGW), jnp.float32),
    mesh=_mesh,
    scratch_types=[
        pltpu.VMEM((DCHUNK, CHUNK), jnp.int32),
        pltpu.VMEM((CHUNK, DEGW), jnp.float32),
        pltpu.SemaphoreType.DMA,
        pltpu.VMEM_SHARED((N_PAD, DEGW), jnp.float32),
    ],
    compiler_params=_sc_params,
)


def _agg_body(table_hbm, src_hbm, dst_hbm, out_hbm,
              src_v, dst_v, rows0, rows1, rows2, rows3,
              g0, g1, g2, g3, s0, s1, s2, s3, acc):
    c = lax.axis_index("c")
    s = lax.axis_index("s")
    rows = [rows0, rows1, rows2, rows3]
    gsem = [g0, g1, g2, g3]
    ssem = [s0, s1, s2, s3]

    _fill_const(rows0, CHUNK, ACC_W, 0.0)
    _zero_stripe(rows0, acc, s, g0)
    plsc.subcore_barrier()

    pltpu.sync_copy(src_hbm.at[s], src_v)
    pltpu.sync_copy(dst_hbm.at[s], dst_v)

    table = table_hbm.at[c]

    def gath(j, b):
        pltpu.async_copy(table.at[src_v.at[j]], rows[b], gsem[b])

    def wait_g(j, b):
        pltpu.make_async_copy(table.at[src_v.at[j]], rows[b], gsem[b]).wait()

    def scat(j, b):
        pltpu.async_copy(rows[b], acc.at[dst_v.at[j]], ssem[b], add=True)

    def wait_s(j, b):
        pltpu.make_async_copy(rows[b], acc.at[dst_v.at[j]], ssem[b]).wait()

    gath(0, 0)
    gath(1, 1)
    wait_g(0, 0)
    scat(0, 0)
    gath(2, 2)
    wait_g(1, 1)
    scat(1, 1)
    gath(3, 3)
    wait_g(2, 2)
    wait_s(0, 0)
    scat(2, 2)
    gath(4, 0)
    wait_g(3, 3)
    wait_s(1, 1)
    scat(3, 3)
    gath(5, 1)

    def body(k, _):
        for b in range(4):
            j = 4 * k + b
            wait_g(j, b)
            wait_s(j - 2, (b + 2) % 4)
            scat(j, b)
            gath(j + 2, (b + 2) % 4)
        return 0

    lax.fori_loop(1, NCHUNK // 4 - 1, body, 0)
    for b in range(4):
        j = NCHUNK - 4 + b
        wait_g(j, b)
        wait_s(j - 2, (b + 2) % 4)
        scat(j, b)
        if j + 2 < NCHUNK:
            gath(j + 2, (b + 2) % 4)
    wait_s(NCHUNK - 2, 2)
    wait_s(NCHUNK - 1, 3)

    plsc.subcore_barrier()
    pltpu.sync_copy(
        acc.at[pl.ds(s * STRIPE, STRIPE)],
        out_hbm.at[c, pl.ds(s * STRIPE, STRIPE)],
    )


_agg_call = pl.kernel(
    _agg_body,
    out_type=jax.ShapeDtypeStruct((NC, N_PAD, ACC_W), jnp.float32),
    mesh=_mesh,
    scratch_types=[
        pltpu.VMEM((NCHUNK, CHUNK), jnp.int32),
        pltpu.VMEM((NCHUNK, CHUNK), jnp.int32),
        pltpu.VMEM((CHUNK, ACC_W), jnp.float32),
        pltpu.VMEM((CHUNK, ACC_W), jnp.float32),
        pltpu.VMEM((CHUNK, ACC_W), jnp.float32),
        pltpu.VMEM((CHUNK, ACC_W), jnp.float32),
        pltpu.SemaphoreType.DMA,
        pltpu.SemaphoreType.DMA,
        pltpu.SemaphoreType.DMA,
        pltpu.SemaphoreType.DMA,
        pltpu.SemaphoreType.DMA,
        pltpu.SemaphoreType.DMA,
        pltpu.SemaphoreType.DMA,
        pltpu.SemaphoreType.DMA,
        pltpu.VMEM_SHARED((N_PAD, ACC_W), jnp.float32),
    ],
    compiler_params=_sc_params,
)


ROWS_BLK = 1000
GRID = N_NODES // ROWS_BLK


def _dis_block(deg_ref):
    d = deg_ref[0, :, 0:1] + deg_ref[1, :, 0:1] + 1.0
    return lax.rsqrt(d)


def _scale_body(deg_ref, x_ref, o_ref):
    dis = _dis_block(deg_ref)
    sc = x_ref[...] * dis
    o_ref[0, :, :] = sc[:, :ACC_W]
    o_ref[1, :, :] = sc[:, ACC_W:]


_scale_call = pl.pallas_call(
    _scale_body,
    grid=(GRID,),
    in_specs=[
        pl.BlockSpec((NC, ROWS_BLK, DEGW), lambda i: (0, i, 0)),
        pl.BlockSpec((ROWS_BLK, IN_CH), lambda i: (i, 0)),
    ],
    out_specs=pl.BlockSpec((NC, ROWS_BLK, ACC_W), lambda i: (0, i, 0)),
    out_shape=jax.ShapeDtypeStruct((NC, N_NODES, ACC_W), jnp.float32),
)


def _mid_body(deg_ref, agg_ref, s1_ref, w1_ref, b1_ref, w2_ref, o_ref):
    dis = _dis_block(deg_ref)
    a_lo = (agg_ref[0, :, :] + s1_ref[0, :, :]) * dis
    a_hi = (agg_ref[1, :, :] + s1_ref[1, :, :]) * dis
    a = jnp.concatenate([a_lo, a_hi], axis=1)
    h = jnp.dot(a, w1_ref[...], preferred_element_type=jnp.float32)
    h = jnp.maximum(h + b1_ref[...], 0.0)
    t = jnp.dot(h, w2_ref[...], preferred_element_type=jnp.float32) * dis
    o_ref[0, :, :] = t[:, :ACC_W]
    o_ref[1, :, :] = t[:, ACC_W:]


_mid_call = pl.pallas_call(
    _mid_body,
    grid=(GRID,),
    in_specs=[
        pl.BlockSpec((NC, ROWS_BLK, DEGW), lambda i: (0, i, 0)),
        pl.BlockSpec((NC, ROWS_BLK, ACC_W), lambda i: (0, i, 0)),
        pl.BlockSpec((NC, ROWS_BLK, ACC_W), lambda i: (0, i, 0)),
        pl.BlockSpec((IN_CH, HID), lambda i: (0, 0)),
        pl.BlockSpec((1, HID), lambda i: (0, 0)),
        pl.BlockSpec((HID, OUT_CH), lambda i: (0, 0)),
    ],
    out_specs=pl.BlockSpec((NC, ROWS_BLK, ACC_W), lambda i: (0, i, 0)),
    out_shape=jax.ShapeDtypeStruct((NC, N_NODES, ACC_W), jnp.float32),
)


def _out_body(deg_ref, agg_ref, s2_ref, b2_ref, o_ref):
    dis = _dis_block(deg_ref)
    o_lo = (agg_ref[0, :, :] + s2_ref[0, :, :]) * dis
    o_hi = (agg_ref[1, :, :] + s2_ref[1, :, :]) * dis
    o_ref[...] = jnp.concatenate([o_lo, o_hi], axis=1) + b2_ref[...]


_out_call = pl.pallas_call(
    _out_body,
    grid=(GRID,),
    in_specs=[
        pl.BlockSpec((NC, ROWS_BLK, DEGW), lambda i: (0, i, 0)),
        pl.BlockSpec((NC, ROWS_BLK, ACC_W), lambda i: (0, i, 0)),
        pl.BlockSpec((NC, ROWS_BLK, ACC_W), lambda i: (0, i, 0)),
        pl.BlockSpec((1, OUT_CH), lambda i: (0, 0)),
    ],
    out_specs=pl.BlockSpec((ROWS_BLK, OUT_CH), lambda i: (i, 0)),
    out_shape=jax.ShapeDtypeStruct((N_NODES, OUT_CH), jnp.float32),
)


@jax.jit
def kernel(x, edge_index, W1, b1, W2, b2):
    ei = edge_index.astype(jnp.int32)
    src2 = ei[0].reshape(NS, NCHUNK, CHUNK)
    dst2 = ei[1].reshape(NS, NCHUNK, CHUNK)

    degp = _deg_call(dst2)
    s1 = _scale_call(degp, x)
    a1 = _agg_call(s1, src2, dst2)
    s2 = _mid_call(degp, a1, s1, W1, b1.reshape(1, HID), W2)
    a2 = _agg_call(s2, src2, dst2)
    return _out_call(degp, a2, s2, b2.reshape(1, OUT_CH))


# speedup vs baseline: 31.7077x; 1.0351x over previous
import jax
import jax.numpy as jnp
from jax import lax
from jax.experimental import pallas as pl
from jax.experimental.pallas import tpu as pltpu
from jax.experimental.pallas import tpu_sc as plsc

N_NODES = 10000
N_PAD = 10240
IN_CH = 128
HID = 256
OUT_CH = 128
N_EDGES = 320000

NC = 2
NS = 16
E_PER_T = N_EDGES // NS
CHUNK = 125
NCHUNK = E_PER_T // CHUNK
DCHUNK = NCHUNK // NC
STRIPE = N_PAD // NS
ZROWS = STRIPE // 16
DEGW = 16
ACC_W = 64

_mesh = plsc.VectorSubcoreMesh(core_axis_name="c", subcore_axis_name="s")
_sc_params = pltpu.CompilerParams(use_tc_tiling_on_sc=False)


def _fill_const(ref, rows, width, value):
    lanes = width // 16

    def body(i, _):
        r = i // lanes
        k = i % lanes
        ref[r, pl.ds(k * 16, 16)] = jnp.full((16,), value, jnp.float32)
        return 0

    lax.fori_loop(0, rows * lanes, body, 0)


def _zero_stripe(zsrc, acc, s, zsem):
    n = STRIPE // ZROWS
    for q in range(n):
        pltpu.async_copy(
            zsrc.at[pl.ds(0, ZROWS)],
            acc.at[pl.ds(s * STRIPE + q * ZROWS, ZROWS)],
            zsem,
        )
    for q in range(n):
        pltpu.make_async_copy(
            zsrc.at[pl.ds(0, ZROWS)],
            acc.at[pl.ds(s * STRIPE, ZROWS)],
            zsem,
        ).wait()


def _deg_body(dst_hbm, out_hbm, dst_v, ones_v, ssem, acc):
    c = lax.axis_index("c")
    s = lax.axis_index("s")

    _fill_const(ones_v, CHUNK, DEGW, 0.0)
    _zero_stripe(ones_v, acc, s, ssem)
    _fill_const(ones_v, CHUNK, DEGW, 1.0)
    plsc.subcore_barrier()

    pltpu.sync_copy(dst_hbm.at[s, pl.ds(c * DCHUNK, DCHUNK)], dst_v)

    depth = 4
    for q in range(depth):
        pltpu.async_copy(ones_v, acc.at[dst_v.at[q]], ssem, add=True)

    def body(i, _):
        pltpu.make_async_copy(ones_v, acc.at[dst_v.at[i]], ssem).wait()
        pltpu.async_copy(ones_v, acc.at[dst_v.at[i + depth]], ssem, add=True)
        return 0

    lax.fori_loop(0, DCHUNK - depth, body, 0)
    for q in range(depth):
        pltpu.make_async_copy(ones_v, acc.at[dst_v.at[q]], ssem).wait()

    plsc.subcore_barrier()
    pltpu.sync_copy(
        acc.at[pl.ds(s * STRIPE, STRIPE)],
        out_hbm.at[c, pl.ds(s * STRIPE, STRIPE)],
    )


_deg_call = pl.kernel(
    _deg_body,
    out_type=jax.ShapeDtypeStruct((NC, N_PAD, DEGW), jnp.float32),
    mesh=_mesh,
    scratch_types=[
        pltpu.VMEM((DCHUNK, CHUNK), jnp.int32),
        pltpu.VMEM((CHUNK, DEGW), jnp.float32),
        pltpu.SemaphoreType.DMA,
        pltpu.VMEM_SHARED((N_PAD, DEGW), jnp.float32),
    ],
    compiler_params=_sc_params,
)


def _agg_body(table_hbm, src_hbm, dst_hbm, out_hbm,
              src_v, dst_v, rows0, rows1, rows2, rows3,
              g0, g1, g2, g3, s0, s1, s2, s3, acc):
    c = lax.axis_index("c")
    s = lax.axis_index("s")
    rows = [rows0, rows1, rows2, rows3]
    gsem = [g0, g1, g2, g3]
    ssem = [s0, s1, s2, s3]

    pltpu.async_copy(src_hbm.at[s], src_v, g1)
    pltpu.async_copy(dst_hbm.at[s], dst_v, g2)
    _fill_const(rows0, CHUNK, ACC_W, 0.0)
    _zero_stripe(rows0, acc, s, g0)
    plsc.subcore_barrier()
    pltpu.make_async_copy(src_hbm.at[s], src_v, g1).wait()
    pltpu.make_async_copy(dst_hbm.at[s], dst_v, g2).wait()

    table = table_hbm.at[c]

    def gath(j, b):
        pltpu.async_copy(table.at[src_v.at[j]], rows[b], gsem[b])

    def wait_g(j, b):
        pltpu.make_async_copy(table.at[src_v.at[j]], rows[b], gsem[b]).wait()

    def scat(j, b):
        pltpu.async_copy(rows[b], acc.at[dst_v.at[j]], ssem[b], add=True)

    def wait_s(j, b):
        pltpu.make_async_copy(rows[b], acc.at[dst_v.at[j]], ssem[b]).wait()

    gath(0, 0)
    gath(1, 1)
    wait_g(0, 0)
    scat(0, 0)
    gath(2, 2)
    wait_g(1, 1)
    scat(1, 1)
    gath(3, 3)
    wait_g(2, 2)
    wait_s(0, 0)
    scat(2, 2)
    gath(4, 0)
    wait_g(3, 3)
    wait_s(1, 1)
    scat(3, 3)
    gath(5, 1)

    def body(k, _):
        for b in range(4):
            j = 4 * k + b
            wait_g(j, b)
            wait_s(j - 2, (b + 2) % 4)
            scat(j, b)
            gath(j + 2, (b + 2) % 4)
        return 0

    lax.fori_loop(1, NCHUNK // 4 - 1, body, 0)
    for b in range(4):
        j = NCHUNK - 4 + b
        wait_g(j, b)
        wait_s(j - 2, (b + 2) % 4)
        scat(j, b)
        if j + 2 < NCHUNK:
            gath(j + 2, (b + 2) % 4)
    wait_s(NCHUNK - 2, 2)
    wait_s(NCHUNK - 1, 3)

    plsc.subcore_barrier()
    pltpu.sync_copy(
        acc.at[pl.ds(s * STRIPE, STRIPE)],
        out_hbm.at[c, pl.ds(s * STRIPE, STRIPE)],
    )


_agg_call = pl.kernel(
    _agg_body,
    out_type=jax.ShapeDtypeStruct((NC, N_PAD, ACC_W), jnp.float32),
    mesh=_mesh,
    scratch_types=[
        pltpu.VMEM((NCHUNK, CHUNK), jnp.int32),
        pltpu.VMEM((NCHUNK, CHUNK), jnp.int32),
        pltpu.VMEM((CHUNK, ACC_W), jnp.float32),
        pltpu.VMEM((CHUNK, ACC_W), jnp.float32),
        pltpu.VMEM((CHUNK, ACC_W), jnp.float32),
        pltpu.VMEM((CHUNK, ACC_W), jnp.float32),
        pltpu.SemaphoreType.DMA,
        pltpu.SemaphoreType.DMA,
        pltpu.SemaphoreType.DMA,
        pltpu.SemaphoreType.DMA,
        pltpu.SemaphoreType.DMA,
        pltpu.SemaphoreType.DMA,
        pltpu.SemaphoreType.DMA,
        pltpu.SemaphoreType.DMA,
        pltpu.VMEM_SHARED((N_PAD, ACC_W), jnp.float32),
    ],
    compiler_params=_sc_params,
)


ROWS_BLK = 2000
GRID = N_NODES // ROWS_BLK


def _dis_block(deg_ref):
    d = deg_ref[0, :, 0:1] + deg_ref[1, :, 0:1] + 1.0
    return lax.rsqrt(d)


def _scale_body(deg_ref, x_ref, o_ref):
    dis = _dis_block(deg_ref)
    sc = x_ref[...] * dis
    o_ref[0, :, :] = sc[:, :ACC_W]
    o_ref[1, :, :] = sc[:, ACC_W:]


_scale_call = pl.pallas_call(
    _scale_body,
    grid=(GRID,),
    in_specs=[
        pl.BlockSpec((NC, ROWS_BLK, DEGW), lambda i: (0, i, 0)),
        pl.BlockSpec((ROWS_BLK, IN_CH), lambda i: (i, 0)),
    ],
    out_specs=pl.BlockSpec((NC, ROWS_BLK, ACC_W), lambda i: (0, i, 0)),
    out_shape=jax.ShapeDtypeStruct((NC, N_NODES, ACC_W), jnp.float32),
)


def _mid_body(deg_ref, agg_ref, s1_ref, w1_ref, b1_ref, w2_ref, o_ref):
    dis = _dis_block(deg_ref)
    a_lo = (agg_ref[0, :, :] + s1_ref[0, :, :]) * dis
    a_hi = (agg_ref[1, :, :] + s1_ref[1, :, :]) * dis
    a = jnp.concatenate([a_lo, a_hi], axis=1)
    h = jnp.dot(a, w1_ref[...], preferred_element_type=jnp.float32)
    h = jnp.maximum(h + b1_ref[...], 0.0)
    t = jnp.dot(h, w2_ref[...], preferred_element_type=jnp.float32) * dis
    o_ref[0, :, :] = t[:, :ACC_W]
    o_ref[1, :, :] = t[:, ACC_W:]


_mid_call = pl.pallas_call(
    _mid_body,
    grid=(GRID,),
    in_specs=[
        pl.BlockSpec((NC, ROWS_BLK, DEGW), lambda i: (0, i, 0)),
        pl.BlockSpec((NC, ROWS_BLK, ACC_W), lambda i: (0, i, 0)),
        pl.BlockSpec((NC, ROWS_BLK, ACC_W), lambda i: (0, i, 0)),
        pl.BlockSpec((IN_CH, HID), lambda i: (0, 0)),
        pl.BlockSpec((1, HID), lambda i: (0, 0)),
        pl.BlockSpec((HID, OUT_CH), lambda i: (0, 0)),
    ],
    out_specs=pl.BlockSpec((NC, ROWS_BLK, ACC_W), lambda i: (0, i, 0)),
    out_shape=jax.ShapeDtypeStruct((NC, N_NODES, ACC_W), jnp.float32),
)


def _out_body(deg_ref, agg_ref, s2_ref, b2_ref, o_ref):
    dis = _dis_block(deg_ref)
    o_lo = (agg_ref[0, :, :] + s2_ref[0, :, :]) * dis
    o_hi = (agg_ref[1, :, :] + s2_ref[1, :, :]) * dis
    o_ref[...] = jnp.concatenate([o_lo, o_hi], axis=1) + b2_ref[...]


_out_call = pl.pallas_call(
    _out_body,
    grid=(GRID,),
    in_specs=[
        pl.BlockSpec((NC, ROWS_BLK, DEGW), lambda i: (0, i, 0)),
        pl.BlockSpec((NC, ROWS_BLK, ACC_W), lambda i: (0, i, 0)),
        pl.BlockSpec((NC, ROWS_BLK, ACC_W), lambda i: (0, i, 0)),
        pl.BlockSpec((1, OUT_CH), lambda i: (0, 0)),
    ],
    out_specs=pl.BlockSpec((ROWS_BLK, OUT_CH), lambda i: (i, 0)),
    out_shape=jax.ShapeDtypeStruct((N_NODES, OUT_CH), jnp.float32),
)


@jax.jit
def kernel(x, edge_index, W1, b1, W2, b2):
    ei = edge_index.astype(jnp.int32)
    src2 = ei[0].reshape(NS, NCHUNK, CHUNK)
    dst2 = ei[1].reshape(NS, NCHUNK, CHUNK)

    degp = _deg_call(dst2)
    s1 = _scale_call(degp, x)
    a1 = _agg_call(s1, src2, dst2)
    s2 = _mid_call(degp, a1, s1, W1, b1.reshape(1, HID), W2)
    a2 = _agg_call(s2, src2, dst2)
    return _out_call(degp, a2, s2, b2.reshape(1, OUT_CH))


# speedup vs baseline: 32.0485x; 1.0107x over previous
import jax
import jax.numpy as jnp
from jax import lax
from jax.experimental import pallas as pl
from jax.experimental.pallas import tpu as pltpu
from jax.experimental.pallas import tpu_sc as plsc

N_NODES = 10000
N_PAD = 10240
IN_CH = 128
HID = 256
OUT_CH = 128
N_EDGES = 320000

NC = 2
NS = 16
E_PER_T = N_EDGES // NS
CHUNK = 125
NCHUNK = E_PER_T // CHUNK
DCHUNK = NCHUNK // NC
STRIPE = N_PAD // NS
ZROWS = STRIPE // 16
DEGW = 16
ACC_W = 64

_mesh = plsc.VectorSubcoreMesh(core_axis_name="c", subcore_axis_name="s")
_sc_params = pltpu.CompilerParams(use_tc_tiling_on_sc=False)


def _fill_const(ref, rows, width, value):
    lanes = width // 16

    def body(i, _):
        r = i // lanes
        k = i % lanes
        ref[r, pl.ds(k * 16, 16)] = jnp.full((16,), value, jnp.float32)
        return 0

    lax.fori_loop(0, rows * lanes, body, 0)


def _zero_stripe(zsrc, acc, s, zsem):
    n = STRIPE // ZROWS
    for q in range(n):
        pltpu.async_copy(
            zsrc.at[pl.ds(0, ZROWS)],
            acc.at[pl.ds(s * STRIPE + q * ZROWS, ZROWS)],
            zsem,
        )
    for q in range(n):
        pltpu.make_async_copy(
            zsrc.at[pl.ds(0, ZROWS)],
            acc.at[pl.ds(s * STRIPE, ZROWS)],
            zsem,
        ).wait()


def _deg_body(dst_hbm, out_hbm, dst_v, ones_v, ssem, zsem, acc):
    c = lax.axis_index("c")
    s = lax.axis_index("s")

    pltpu.async_copy(dst_hbm.at[s, pl.ds(c * DCHUNK, DCHUNK)], dst_v, ssem)
    _fill_const(ones_v, CHUNK, DEGW, 0.0)
    _zero_stripe(ones_v, acc, s, zsem)
    _fill_const(ones_v, CHUNK, DEGW, 1.0)
    plsc.subcore_barrier()
    pltpu.make_async_copy(
        dst_hbm.at[s, pl.ds(c * DCHUNK, DCHUNK)], dst_v, ssem
    ).wait()

    depth = 4
    for q in range(depth):
        pltpu.async_copy(ones_v, acc.at[dst_v.at[q]], ssem, add=True)

    def body(i, _):
        pltpu.make_async_copy(ones_v, acc.at[dst_v.at[i]], ssem).wait()
        pltpu.async_copy(ones_v, acc.at[dst_v.at[i + depth]], ssem, add=True)
        return 0

    lax.fori_loop(0, DCHUNK - depth, body, 0)
    for q in range(depth):
        pltpu.make_async_copy(ones_v, acc.at[dst_v.at[q]], ssem).wait()

    plsc.subcore_barrier()
    pltpu.sync_copy(
        acc.at[pl.ds(s * STRIPE, STRIPE)],
        out_hbm.at[c, pl.ds(s * STRIPE, STRIPE)],
    )


_deg_call = pl.kernel(
    _deg_body,
    out_type=jax.ShapeDtypeStruct((NC, N_PAD, DEGW), jnp.float32),
    mesh=_mesh,
    scratch_types=[
        pltpu.VMEM((DCHUNK, CHUNK), jnp.int32),
        pltpu.VMEM((CHUNK, DEGW), jnp.float32),
        pltpu.SemaphoreType.DMA,
        pltpu.SemaphoreType.DMA,
        pltpu.VMEM_SHARED((N_PAD, DEGW), jnp.float32),
    ],
    compiler_params=_sc_params,
)


def _agg_body(table_hbm, src_hbm, dst_hbm, out_hbm,
              src_v, dst_v, rows0, rows1, rows2, rows3,
              g0, g1, g2, g3, s0, s1, s2, s3, acc):
    c = lax.axis_index("c")
    s = lax.axis_index("s")
    rows = [rows0, rows1, rows2, rows3]
    gsem = [g0, g1, g2, g3]
    ssem = [s0, s1, s2, s3]

    pltpu.async_copy(src_hbm.at[s], src_v, g1)
    pltpu.async_copy(dst_hbm.at[s], dst_v, g2)
    _fill_const(rows0, CHUNK, ACC_W, 0.0)
    _zero_stripe(rows0, acc, s, g0)
    plsc.subcore_barrier()
    pltpu.make_async_copy(src_hbm.at[s], src_v, g1).wait()
    pltpu.make_async_copy(dst_hbm.at[s], dst_v, g2).wait()

    table = table_hbm.at[c]

    def gath(j, b):
        pltpu.async_copy(table.at[src_v.at[j]], rows[b], gsem[b])

    def wait_g(j, b):
        pltpu.make_async_copy(table.at[src_v.at[j]], rows[b], gsem[b]).wait()

    def scat(j, b):
        pltpu.async_copy(rows[b], acc.at[dst_v.at[j]], ssem[b], add=True)

    def wait_s(j, b):
        pltpu.make_async_copy(rows[b], acc.at[dst_v.at[j]], ssem[b]).wait()

    gath(0, 0)
    gath(1, 1)
    wait_g(0, 0)
    scat(0, 0)
    gath(2, 2)
    wait_g(1, 1)
    scat(1, 1)
    gath(3, 3)
    wait_g(2, 2)
    wait_s(0, 0)
    scat(2, 2)
    gath(4, 0)
    wait_g(3, 3)
    wait_s(1, 1)
    scat(3, 3)
    gath(5, 1)

    def body(k, _):
        for b in range(4):
            j = 4 * k + b
            wait_g(j, b)
            wait_s(j - 2, (b + 2) % 4)
            scat(j, b)
            gath(j + 2, (b + 2) % 4)
        return 0

    lax.fori_loop(1, NCHUNK // 4 - 1, body, 0)
    for b in range(4):
        j = NCHUNK - 4 + b
        wait_g(j, b)
        wait_s(j - 2, (b + 2) % 4)
        scat(j, b)
        if j + 2 < NCHUNK:
            gath(j + 2, (b + 2) % 4)
    wait_s(NCHUNK - 2, 2)
    wait_s(NCHUNK - 1, 3)

    plsc.subcore_barrier()
    pltpu.sync_copy(
        acc.at[pl.ds(s * STRIPE, STRIPE)],
        out_hbm.at[c, pl.ds(s * STRIPE, STRIPE)],
    )


_agg_call = pl.kernel(
    _agg_body,
    out_type=jax.ShapeDtypeStruct((NC, N_PAD, ACC_W), jnp.float32),
    mesh=_mesh,
    scratch_types=[
        pltpu.VMEM((NCHUNK, CHUNK), jnp.int32),
        pltpu.VMEM((NCHUNK, CHUNK), jnp.int32),
        pltpu.VMEM((CHUNK, ACC_W), jnp.float32),
        pltpu.VMEM((CHUNK, ACC_W), jnp.float32),
        pltpu.VMEM((CHUNK, ACC_W), jnp.float32),
        pltpu.VMEM((CHUNK, ACC_W), jnp.float32),
        pltpu.SemaphoreType.DMA,
        pltpu.SemaphoreType.DMA,
        pltpu.SemaphoreType.DMA,
        pltpu.SemaphoreType.DMA,
        pltpu.SemaphoreType.DMA,
        pltpu.SemaphoreType.DMA,
        pltpu.SemaphoreType.DMA,
        pltpu.SemaphoreType.DMA,
        pltpu.VMEM_SHARED((N_PAD, ACC_W), jnp.float32),
    ],
    compiler_params=_sc_params,
)


ROWS_BLK = 5000
GRID = N_NODES // ROWS_BLK


def _dis_block(deg_ref):
    d = deg_ref[0, :, 0:1] + deg_ref[1, :, 0:1] + 1.0
    return lax.rsqrt(d)


def _scale_body(deg_ref, x_ref, o_ref):
    dis = _dis_block(deg_ref)
    sc = x_ref[...] * dis
    o_ref[0, :, :] = sc[:, :ACC_W]
    o_ref[1, :, :] = sc[:, ACC_W:]


_scale_call = pl.pallas_call(
    _scale_body,
    grid=(GRID,),
    in_specs=[
        pl.BlockSpec((NC, ROWS_BLK, DEGW), lambda i: (0, i, 0)),
        pl.BlockSpec((ROWS_BLK, IN_CH), lambda i: (i, 0)),
    ],
    out_specs=pl.BlockSpec((NC, ROWS_BLK, ACC_W), lambda i: (0, i, 0)),
    out_shape=jax.ShapeDtypeStruct((NC, N_NODES, ACC_W), jnp.float32),
)


def _mid_body(deg_ref, agg_ref, s1_ref, w1_ref, b1_ref, w2_ref, o_ref):
    dis = _dis_block(deg_ref)
    a_lo = (agg_ref[0, :, :] + s1_ref[0, :, :]) * dis
    a_hi = (agg_ref[1, :, :] + s1_ref[1, :, :]) * dis
    a = jnp.concatenate([a_lo, a_hi], axis=1)
    h = jnp.dot(a, w1_ref[...], preferred_element_type=jnp.float32)
    h = jnp.maximum(h + b1_ref[...], 0.0)
    t = jnp.dot(h, w2_ref[...], preferred_element_type=jnp.float32) * dis
    o_ref[0, :, :] = t[:, :ACC_W]
    o_ref[1, :, :] = t[:, ACC_W:]


_mid_call = pl.pallas_call(
    _mid_body,
    grid=(GRID,),
    in_specs=[
        pl.BlockSpec((NC, ROWS_BLK, DEGW), lambda i: (0, i, 0)),
        pl.BlockSpec((NC, ROWS_BLK, ACC_W), lambda i: (0, i, 0)),
        pl.BlockSpec((NC, ROWS_BLK, ACC_W), lambda i: (0, i, 0)),
        pl.BlockSpec((IN_CH, HID), lambda i: (0, 0)),
        pl.BlockSpec((1, HID), lambda i: (0, 0)),
        pl.BlockSpec((HID, OUT_CH), lambda i: (0, 0)),
    ],
    out_specs=pl.BlockSpec((NC, ROWS_BLK, ACC_W), lambda i: (0, i, 0)),
    out_shape=jax.ShapeDtypeStruct((NC, N_NODES, ACC_W), jnp.float32),
)


def _out_body(deg_ref, agg_ref, s2_ref, b2_ref, o_ref):
    dis = _dis_block(deg_ref)
    o_lo = (agg_ref[0, :, :] + s2_ref[0, :, :]) * dis
    o_hi = (agg_ref[1, :, :] + s2_ref[1, :, :]) * dis
    o_ref[...] = jnp.concatenate([o_lo, o_hi], axis=1) + b2_ref[...]


_out_call = pl.pallas_call(
    _out_body,
    grid=(GRID,),
    in_specs=[
        pl.BlockSpec((NC, ROWS_BLK, DEGW), lambda i: (0, i, 0)),
        pl.BlockSpec((NC, ROWS_BLK, ACC_W), lambda i: (0, i, 0)),
        pl.BlockSpec((NC, ROWS_BLK, ACC_W), lambda i: (0, i, 0)),
        pl.BlockSpec((1, OUT_CH), lambda i: (0, 0)),
    ],
    out_specs=pl.BlockSpec((ROWS_BLK, OUT_CH), lambda i: (i, 0)),
    out_shape=jax.ShapeDtypeStruct((N_NODES, OUT_CH), jnp.float32),
)


@jax.jit
def kernel(x, edge_index, W1, b1, W2, b2):
    ei = edge_index.astype(jnp.int32)
    src2 = ei[0].reshape(NS, NCHUNK, CHUNK)
    dst2 = ei[1].reshape(NS, NCHUNK, CHUNK)

    degp = _deg_call(dst2)
    s1 = _scale_call(degp, x)
    a1 = _agg_call(s1, src2, dst2)
    s2 = _mid_call(degp, a1, s1, W1, b1.reshape(1, HID), W2)
    a2 = _agg_call(s2, src2, dst2)
    return _out_call(degp, a2, s2, b2.reshape(1, OUT_CH))


# speedup vs baseline: 32.8035x; 1.0236x over previous
import jax
import jax.numpy as jnp
from jax import lax
from jax.experimental import pallas as pl
from jax.experimental.pallas import tpu as pltpu
from jax.experimental.pallas import tpu_sc as plsc

N_NODES = 10000
N_PAD = 10240
IN_CH = 128
HID = 256
OUT_CH = 128
N_EDGES = 320000

NC = 2
NS = 16
E_PER_T = N_EDGES // NS
CHUNK = 125
NCHUNK = E_PER_T // CHUNK
DCHUNK = NCHUNK // NC
STRIPE = N_PAD // NS
ZROWS = STRIPE // 16
DEGW = 16
ACC_W = 64

_mesh = plsc.VectorSubcoreMesh(core_axis_name="c", subcore_axis_name="s")
_sc_params = pltpu.CompilerParams(use_tc_tiling_on_sc=False)


def _fill_const(ref, rows, width, value):
    lanes = width // 16

    def body(i, _):
        r = i // lanes
        k = i % lanes
        ref[r, pl.ds(k * 16, 16)] = jnp.full((16,), value, jnp.float32)
        return 0

    lax.fori_loop(0, rows * lanes, body, 0)


def _zero_stripe(zsrc, acc, s, zsem):
    n = STRIPE // ZROWS
    for q in range(n):
        pltpu.async_copy(
            zsrc.at[pl.ds(0, ZROWS)],
            acc.at[pl.ds(s * STRIPE + q * ZROWS, ZROWS)],
            zsem,
        )
    for q in range(n):
        pltpu.make_async_copy(
            zsrc.at[pl.ds(0, ZROWS)],
            acc.at[pl.ds(s * STRIPE, ZROWS)],
            zsem,
        ).wait()


def _deg_body(idx_hbm, out_hbm, dst_v, ones_v, ssem, zsem, acc):
    c = lax.axis_index("c")
    s = lax.axis_index("s")

    pltpu.async_copy(idx_hbm.at[1, s, pl.ds(c * DCHUNK, DCHUNK)], dst_v, ssem)
    _fill_const(ones_v, CHUNK, DEGW, 0.0)
    _zero_stripe(ones_v, acc, s, zsem)
    _fill_const(ones_v, CHUNK, DEGW, 1.0)
    plsc.subcore_barrier()
    pltpu.make_async_copy(
        idx_hbm.at[1, s, pl.ds(c * DCHUNK, DCHUNK)], dst_v, ssem
    ).wait()

    depth = 4
    for q in range(depth):
        pltpu.async_copy(ones_v, acc.at[dst_v.at[q]], ssem, add=True)

    def body(i, _):
        pltpu.make_async_copy(ones_v, acc.at[dst_v.at[i]], ssem).wait()
        pltpu.async_copy(ones_v, acc.at[dst_v.at[i + depth]], ssem, add=True)
        return 0

    lax.fori_loop(0, DCHUNK - depth, body, 0)
    for q in range(depth):
        pltpu.make_async_copy(ones_v, acc.at[dst_v.at[q]], ssem).wait()

    plsc.subcore_barrier()
    pltpu.sync_copy(
        acc.at[pl.ds(s * STRIPE, STRIPE)],
        out_hbm.at[c, pl.ds(s * STRIPE, STRIPE)],
    )


_deg_call = pl.kernel(
    _deg_body,
    out_type=jax.ShapeDtypeStruct((NC, N_PAD, DEGW), jnp.float32),
    mesh=_mesh,
    scratch_types=[
        pltpu.VMEM((DCHUNK, CHUNK), jnp.int32),
        pltpu.VMEM((CHUNK, DEGW), jnp.float32),
        pltpu.SemaphoreType.DMA,
        pltpu.SemaphoreType.DMA,
        pltpu.VMEM_SHARED((N_PAD, DEGW), jnp.float32),
    ],
    compiler_params=_sc_params,
)


def _agg_body(table_hbm, idx_hbm, out_hbm,
              src_v, dst_v, rows0, rows1, rows2, rows3,
              g0, g1, g2, g3, s0, s1, s2, s3, acc):
    c = lax.axis_index("c")
    s = lax.axis_index("s")
    rows = [rows0, rows1, rows2, rows3]
    gsem = [g0, g1, g2, g3]
    ssem = [s0, s1, s2, s3]

    pltpu.async_copy(idx_hbm.at[0, s], src_v, g1)
    pltpu.async_copy(idx_hbm.at[1, s], dst_v, g2)
    _fill_const(rows0, CHUNK, ACC_W, 0.0)
    _zero_stripe(rows0, acc, s, g0)
    plsc.subcore_barrier()
    pltpu.make_async_copy(idx_hbm.at[0, s], src_v, g1).wait()
    pltpu.make_async_copy(idx_hbm.at[1, s], dst_v, g2).wait()

    table = table_hbm.at[c]

    def gath(j, b):
        pltpu.async_copy(table.at[src_v.at[j]], rows[b], gsem[b])

    def wait_g(j, b):
        pltpu.make_async_copy(table.at[src_v.at[j]], rows[b], gsem[b]).wait()

    def scat(j, b):
        pltpu.async_copy(rows[b], acc.at[dst_v.at[j]], ssem[b], add=True)

    def wait_s(j, b):
        pltpu.make_async_copy(rows[b], acc.at[dst_v.at[j]], ssem[b]).wait()

    gath(0, 0)
    gath(1, 1)
    wait_g(0, 0)
    scat(0, 0)
    gath(2, 2)
    wait_g(1, 1)
    scat(1, 1)
    gath(3, 3)
    wait_g(2, 2)
    wait_s(0, 0)
    scat(2, 2)
    gath(4, 0)
    wait_g(3, 3)
    wait_s(1, 1)
    scat(3, 3)
    gath(5, 1)

    def body(k, _):
        for b in range(4):
            j = 4 * k + b
            wait_g(j, b)
            wait_s(j - 2, (b + 2) % 4)
            scat(j, b)
            gath(j + 2, (b + 2) % 4)
        return 0

    lax.fori_loop(1, NCHUNK // 4 - 1, body, 0)
    for b in range(4):
        j = NCHUNK - 4 + b
        wait_g(j, b)
        wait_s(j - 2, (b + 2) % 4)
        scat(j, b)
        if j + 2 < NCHUNK:
            gath(j + 2, (b + 2) % 4)
    wait_s(NCHUNK - 2, 2)
    wait_s(NCHUNK - 1, 3)

    plsc.subcore_barrier()
    pltpu.sync_copy(
        acc.at[pl.ds(s * STRIPE, STRIPE)],
        out_hbm.at[c, pl.ds(s * STRIPE, STRIPE)],
    )


_agg_call = pl.kernel(
    _agg_body,
    out_type=jax.ShapeDtypeStruct((NC, N_PAD, ACC_W), jnp.float32),
    mesh=_mesh,
    scratch_types=[
        pltpu.VMEM((NCHUNK, CHUNK), jnp.int32),
        pltpu.VMEM((NCHUNK, CHUNK), jnp.int32),
        pltpu.VMEM((CHUNK, ACC_W), jnp.float32),
        pltpu.VMEM((CHUNK, ACC_W), jnp.float32),
        pltpu.VMEM((CHUNK, ACC_W), jnp.float32),
        pltpu.VMEM((CHUNK, ACC_W), jnp.float32),
        pltpu.SemaphoreType.DMA,
        pltpu.SemaphoreType.DMA,
        pltpu.SemaphoreType.DMA,
        pltpu.SemaphoreType.DMA,
        pltpu.SemaphoreType.DMA,
        pltpu.SemaphoreType.DMA,
        pltpu.SemaphoreType.DMA,
        pltpu.SemaphoreType.DMA,
        pltpu.VMEM_SHARED((N_PAD, ACC_W), jnp.float32),
    ],
    compiler_params=_sc_params,
)


ROWS_BLK = 5000
GRID = N_NODES // ROWS_BLK


def _dis_block(deg_ref):
    d = deg_ref[0, :, 0:1] + deg_ref[1, :, 0:1] + 1.0
    return lax.rsqrt(d)


def _scale_body(deg_ref, x_ref, o_ref):
    dis = _dis_block(deg_ref)
    sc = x_ref[...] * dis
    o_ref[0, :, :] = sc[:, :ACC_W]
    o_ref[1, :, :] = sc[:, ACC_W:]


_scale_call = pl.pallas_call(
    _scale_body,
    grid=(GRID,),
    in_specs=[
        pl.BlockSpec((NC, ROWS_BLK, DEGW), lambda i: (0, i, 0)),
        pl.BlockSpec((ROWS_BLK, IN_CH), lambda i: (i, 0)),
    ],
    out_specs=pl.BlockSpec((NC, ROWS_BLK, ACC_W), lambda i: (0, i, 0)),
    out_shape=jax.ShapeDtypeStruct((NC, N_NODES, ACC_W), jnp.float32),
)


def _mid_body(deg_ref, agg_ref, s1_ref, w1_ref, b1_ref, w2_ref, o_ref):
    dis = _dis_block(deg_ref)
    a_lo = (agg_ref[0, :, :] + s1_ref[0, :, :]) * dis
    a_hi = (agg_ref[1, :, :] + s1_ref[1, :, :]) * dis
    a = jnp.concatenate([a_lo, a_hi], axis=1)
    h = jnp.dot(a, w1_ref[...], preferred_element_type=jnp.float32)
    h = jnp.maximum(h + b1_ref[...], 0.0)
    t = jnp.dot(h, w2_ref[...], preferred_element_type=jnp.float32) * dis
    o_ref[0, :, :] = t[:, :ACC_W]
    o_ref[1, :, :] = t[:, ACC_W:]


_mid_call = pl.pallas_call(
    _mid_body,
    grid=(GRID,),
    in_specs=[
        pl.BlockSpec((NC, ROWS_BLK, DEGW), lambda i: (0, i, 0)),
        pl.BlockSpec((NC, ROWS_BLK, ACC_W), lambda i: (0, i, 0)),
        pl.BlockSpec((NC, ROWS_BLK, ACC_W), lambda i: (0, i, 0)),
        pl.BlockSpec((IN_CH, HID), lambda i: (0, 0)),
        pl.BlockSpec((1, HID), lambda i: (0, 0)),
        pl.BlockSpec((HID, OUT_CH), lambda i: (0, 0)),
    ],
    out_specs=pl.BlockSpec((NC, ROWS_BLK, ACC_W), lambda i: (0, i, 0)),
    out_shape=jax.ShapeDtypeStruct((NC, N_NODES, ACC_W), jnp.float32),
)


def _out_body(deg_ref, agg_ref, s2_ref, b2_ref, o_ref):
    dis = _dis_block(deg_ref)
    o_lo = (agg_ref[0, :, :] + s2_ref[0, :, :]) * dis
    o_hi = (agg_ref[1, :, :] + s2_ref[1, :, :]) * dis
    o_ref[...] = jnp.concatenate([o_lo, o_hi], axis=1) + b2_ref[...]


_out_call = pl.pallas_call(
    _out_body,
    grid=(GRID,),
    in_specs=[
        pl.BlockSpec((NC, ROWS_BLK, DEGW), lambda i: (0, i, 0)),
        pl.BlockSpec((NC, ROWS_BLK, ACC_W), lambda i: (0, i, 0)),
        pl.BlockSpec((NC, ROWS_BLK, ACC_W), lambda i: (0, i, 0)),
        pl.BlockSpec((1, OUT_CH), lambda i: (0, 0)),
    ],
    out_specs=pl.BlockSpec((ROWS_BLK, OUT_CH), lambda i: (i, 0)),
    out_shape=jax.ShapeDtypeStruct((N_NODES, OUT_CH), jnp.float32),
)


@jax.jit
def kernel(x, edge_index, W1, b1, W2, b2):
    idx4 = edge_index.astype(jnp.int32).reshape(2, NS, NCHUNK, CHUNK)

    degp = _deg_call(idx4)
    s1 = _scale_call(degp, x)
    a1 = _agg_call(s1, idx4)
    s2 = _mid_call(degp, a1, s1, W1, b1.reshape(1, HID), W2)
    a2 = _agg_call(s2, idx4)
    return _out_call(degp, a2, s2, b2.reshape(1, OUT_CH))
